# baseline (speedup 1.0000x reference)
import functools

import jax
import jax.numpy as jnp
from jax import lax
from jax.experimental import pallas as pl
from jax.experimental.pallas import tpu as pltpu
from jax.experimental.pallas import tpu_sc as plsc

_N = 10000
_E = 320000
_NODE_IN = 128
_EDGE_IN = 16
_ND = 128
_ED = 64

_NW = 32
_EPW = _E // _NW
_CH = 80
_NCH = _EPW // _CH
_NP = 10240
_RPT = _NP // 16

_F32 = jnp.float32


def _mesh():
    return plsc.VectorSubcoreMesh(core_axis_name="c", subcore_axis_name="s")


def _sc_gather(tbl, src3, dst3):
    nw, nch, ch = src3.shape
    epw = nch * ch
    ne = nw * epw

    @functools.partial(
        pl.kernel,
        mesh=_mesh(),
        out_type=[jax.ShapeDtypeStruct((ne, _ND), _F32),
                  jax.ShapeDtypeStruct((ne, _ND), _F32)],
        scratch_types=(
            [pltpu.VMEM((nch, ch), jnp.int32)] * 2
            + [pltpu.VMEM((ch, _ND), _F32)] * 6
            + [pltpu.SemaphoreType.DMA] * 12),
    )
    def k(tbl_hbm, src_hbm, dst_hbm, gs_hbm, gd_hbm,
          sidx, didx, bs0, bs1, bs2, bd0, bd1, bd2,
          g0, g1, g2, h0, h1, h2, w0, w1, w2, x0, x1, x2):
        wid = lax.axis_index("s") * 2 + lax.axis_index("c")
        pltpu.sync_copy(src_hbm.at[wid], sidx)
        pltpu.sync_copy(dst_hbm.at[wid], didx)
        bss, bds = (bs0, bs1, bs2), (bd0, bd1, bd2)
        gss, gds = (g0, g1, g2), (h0, h1, h2)
        wss, wds = (w0, w1, w2), (x0, x1, x2)

        def issue(c, j):
            pltpu.async_copy(tbl_hbm.at[sidx.at[c]], bss[j], gss[j])
            pltpu.async_copy(tbl_hbm.at[didx.at[c]], bds[j], gds[j])

        def wait_g(c, j):
            pltpu.make_async_copy(tbl_hbm.at[sidx.at[c]], bss[j],
                                  gss[j]).wait()
            pltpu.make_async_copy(tbl_hbm.at[didx.at[c]], bds[j],
                                  gds[j]).wait()

        def wr(c, j):
            base = wid * epw + c * ch
            pltpu.async_copy(bss[j], gs_hbm.at[pl.ds(base, ch)], wss[j])
            pltpu.async_copy(bds[j], gd_hbm.at[pl.ds(base, ch)], wds[j])

        def wait_w(c, j):
            base = wid * epw + c * ch
            pltpu.make_async_copy(bss[j], gs_hbm.at[pl.ds(base, ch)],
                                  wss[j]).wait()
            pltpu.make_async_copy(bds[j], gd_hbm.at[pl.ds(base, ch)],
                                  wds[j]).wait()

        issue(0, 0)
        issue(1, 1)
        wait_g(0, 0)
        wr(0, 0)
        issue(2, 2)
        wait_g(1, 1)
        wr(1, 1)
        wait_w(0, 0)
        issue(3, 0)

        def body(g, carry):
            for i in range(3):
                c = 3 * g + 2 + i
                j = (2 + i) % 3
                j2 = (1 + i) % 3

                wait_g(c, j)
                wr(c, j)

                @pl.when(c + 2 < nch)
                def _():
                    wait_w(c - 1, j2)
                    issue(c + 2, j2)

            return carry

        nloop = (nch - 2) // 3
        lax.fori_loop(0, nloop, body, 0)
        for c in range(3 * nloop + 2, nch):
            wait_g(c, c % 3)
            wr(c, c % 3)
        for c in range(nch - 3, nch):
            wait_w(c, c % 3)

    return k(tbl, src3, dst3)


def _sc_segment_sum(vals, dst3, zeros):
    nw, nch, ch = dst3.shape
    epw = nch * ch

    @functools.partial(
        pl.kernel,
        mesh=_mesh(),
        out_type=jax.ShapeDtypeStruct((2 * _NP, _ND), _F32),
        scratch_types=[pltpu.VMEM((nch, ch), jnp.int32),
                       pltpu.VMEM((ch, _ND), _F32),
                       pltpu.VMEM((ch, _ND), _F32),
                       pltpu.VMEM_SHARED((_NP, _ND), _F32),
                       pltpu.SemaphoreType.DMA,
                       pltpu.SemaphoreType.DMA],
    )
    def k(vals_hbm, dst_hbm, z_hbm, out_hbm, didx, bv0, bv1, acc, v0, v1):
        cid = lax.axis_index("c")
        sid = lax.axis_index("s")
        wid = sid * 2 + cid
        r0 = sid * _RPT
        pltpu.sync_copy(z_hbm.at[pl.ds(r0, _RPT)], acc.at[pl.ds(r0, _RPT)])
        pltpu.sync_copy(dst_hbm.at[wid], didx)
        plsc.subcore_barrier()
        bvs, vs = (bv0, bv1), (v0, v1)

        for j in range(2):
            pltpu.async_copy(
                vals_hbm.at[pl.ds(wid * epw + j * ch, ch)], bvs[j], vs[j])

        def step(c, j):
            base = wid * epw + c * ch
            pltpu.make_async_copy(
                vals_hbm.at[pl.ds(base, ch)], bvs[j], vs[j]).wait()
            pltpu.sync_copy(bvs[j], acc.at[didx.at[c]], add=True)

            @pl.when(c + 2 < nch)
            def _():
                pltpu.async_copy(
                    vals_hbm.at[pl.ds(base + 2 * ch, ch)], bvs[j], vs[j])

        def body(g, carry):
            step(2 * g, 0)
            step(2 * g + 1, 1)
            return carry

        lax.fori_loop(0, nch // 2, body, 0)
        if nch % 2:
            step(nch - 1, 0)
        plsc.subcore_barrier()
        pltpu.sync_copy(acc.at[pl.ds(r0, _RPT)],
                        out_hbm.at[pl.ds(cid * _NP + r0, _RPT)])

    return k(vals, dst3, zeros)


_BN = 2000
_BE = 4000


def _full(shape):
    return pl.BlockSpec(shape, lambda i: tuple(0 for _ in shape))


def _rows(block, width):
    return pl.BlockSpec((block, width), lambda i: (i, 0))


def _node_embed(x, Wne, bne, Wsd):
    def body(x_ref, w_ref, b_ref, wsd_ref, nf_ref, t_ref):
        nf = jnp.dot(x_ref[...], w_ref[...],
                     preferred_element_type=_F32) + b_ref[...]
        nf_ref[...] = nf
        t_ref[...] = jnp.dot(nf, wsd_ref[...], preferred_element_type=_F32)

    return pl.pallas_call(
        body,
        grid=(_N // _BN,),
        in_specs=[_rows(_BN, _NODE_IN), _full((_NODE_IN, _ND)),
                  _full((1, _ND)), _full((_ND, _ND))],
        out_specs=[_rows(_BN, _ND), _rows(_BN, _ND)],
        out_shape=[jax.ShapeDtypeStruct((_N, _ND), _F32),
                   jax.ShapeDtypeStruct((_N, _ND), _F32)],
    )(x, Wne, bne.reshape(1, _ND), Wsd)


def _edge_embed(ea, Wee, bee, off, ne):
    ob = off // _BE

    def body(a_ref, w_ref, b_ref, o_ref):
        o_ref[...] = jnp.dot(a_ref[...], w_ref[...],
                             preferred_element_type=_F32) + b_ref[...]

    return pl.pallas_call(
        body,
        grid=(ne // _BE,),
        in_specs=[pl.BlockSpec((_BE, _EDGE_IN), lambda i: (i + ob, 0)),
                  _full((_EDGE_IN, _ED)), _full((1, _ED))],
        out_specs=_rows(_BE, _ED),
        out_shape=jax.ShapeDtypeStruct((ne, _ED), _F32),
    )(ea, Wee, bee.reshape(1, _ED))


def _edge_mlp(gs, gd, ef, W1e, b1, W2, b2, out_pad):
    ne, ef_w = ef.shape
    out_w = _ND if out_pad else _ED

    def body(gs_ref, gd_ref, ef_ref, w1_ref, b1_ref, w2_ref, b2_ref, o_ref):
        efv = ef_ref[...] if ef_w == _ED else ef_ref[:, :_ED]
        h = (gs_ref[:, :_ED] + gd_ref[:, _ED:]
             + jnp.dot(efv, w1_ref[...], preferred_element_type=_F32)
             + b1_ref[...])
        h = jnp.maximum(h, 0.0)
        o = jnp.dot(h, w2_ref[...], preferred_element_type=_F32) + b2_ref[...]
        o = jnp.maximum(o, 0.0)
        if out_pad:
            o_ref[...] = jnp.concatenate(
                [o, jnp.zeros((o.shape[0], _ND - _ED), _F32)], axis=1)
        else:
            o_ref[...] = o

    return pl.pallas_call(
        body,
        grid=(ne // _BE,),
        in_specs=[_rows(_BE, _ND), _rows(_BE, _ND), _rows(_BE, ef_w),
                  _full((_ED, _ED)), _full((1, _ED)),
                  _full((_ED, _ED)), _full((1, _ED))],
        out_specs=_rows(_BE, out_w),
        out_shape=jax.ShapeDtypeStruct((ne, out_w), _F32),
    )(gs, gd, ef, W1e, b1.reshape(1, _ED), W2, b2.reshape(1, _ED))


def _edge_mlp_head(gs, gd, ef, W1e, b1, W2, b2, We1, be1, We2, be2):
    ne, ef_w = ef.shape
    dh = We1.shape[1]

    def body(gs_ref, gd_ref, ef_ref, w1_ref, b1_ref, w2_ref, b2_ref,
             we1_ref, be1_ref, we2_ref, be2_ref, o_ref):
        efv = ef_ref[...] if ef_w == _ED else ef_ref[:, :_ED]
        h = (gs_ref[:, :_ED] + gd_ref[:, _ED:]
             + jnp.dot(efv, w1_ref[...], preferred_element_type=_F32)
             + b1_ref[...])
        h = jnp.maximum(h, 0.0)
        o = jnp.dot(h, w2_ref[...], preferred_element_type=_F32) + b2_ref[...]
        o = jnp.maximum(o, 0.0)
        ch = jnp.maximum(
            jnp.dot(o, we1_ref[...], preferred_element_type=_F32)
            + be1_ref[...], 0.0)
        o_ref[...] = (jnp.dot(ch, we2_ref[...], preferred_element_type=_F32)
                      + be2_ref[...])

    return pl.pallas_call(
        body,
        grid=(ne // _BE,),
        in_specs=[_rows(_BE, _ND), _rows(_BE, _ND), _rows(_BE, ef_w),
                  _full((_ED, _ED)), _full((1, _ED)),
                  _full((_ED, _ED)), _full((1, _ED)),
                  _full((_ED, dh)), _full((1, dh)),
                  _full((dh, 1)), _full((1, 1))],
        out_specs=_rows(_BE, 1),
        out_shape=jax.ShapeDtypeStruct((ne, 1), _F32),
    )(gs, gd, ef, W1e, b1.reshape(1, _ED), W2, b2.reshape(1, _ED),
      We1, be1.reshape(1, dh), We2, be2.reshape(1, 1))


def _node_update(nf, aggs, Wn1, Wn2, bn, Wsd, head=None):
    na = len(aggs)
    agg_specs = [_rows(_BN, _ND)] * na

    def agg_sum(arefs):
        s = arefs[0][:, :_ED]
        for a in arefs[1:]:
            s = s + a[:, :_ED]
        return s

    if head is None:
        def body(*refs):
            nf_ref = refs[0]
            arefs = refs[1:1 + na]
            wn1_ref, wn2_ref, b_ref, wsd_ref, nf2_ref, t_ref = refs[1 + na:]
            h = (jnp.dot(nf_ref[...], wn1_ref[...],
                         preferred_element_type=_F32)
                 + jnp.dot(agg_sum(arefs), wn2_ref[...],
                           preferred_element_type=_F32)
                 + b_ref[...])
            nf2 = jnp.maximum(h, 0.0)
            nf2_ref[...] = nf2
            t_ref[...] = jnp.dot(nf2, wsd_ref[...],
                                 preferred_element_type=_F32)

        return pl.pallas_call(
            body,
            grid=(_N // _BN,),
            in_specs=[_rows(_BN, _ND)] + agg_specs
            + [_full((_ND, _ND)), _full((_ED, _ND)), _full((1, _ND)),
               _full((_ND, _ND))],
            out_specs=[_rows(_BN, _ND), _rows(_BN, _ND)],
            out_shape=[jax.ShapeDtypeStruct((_N, _ND), _F32),
                       jax.ShapeDtypeStruct((_N, _ND), _F32)],
        )(nf, *aggs, Wn1, Wn2, bn.reshape(1, _ND), Wsd)

    Wc1, bc1, Wc2, bc2 = head
    dh = Wc1.shape[1]

    def body(*refs):
        nf_ref = refs[0]
        arefs = refs[1:1 + na]
        (wn1_ref, wn2_ref, b_ref, wsd_ref, wc1_ref, bc1_ref, wc2_ref,
         bc2_ref, t_ref, p_ref) = refs[1 + na:]
        h = (jnp.dot(nf_ref[...], wn1_ref[...], preferred_element_type=_F32)
             + jnp.dot(agg_sum(arefs), wn2_ref[...],
                       preferred_element_type=_F32)
             + b_ref[...])
        nf2 = jnp.maximum(h, 0.0)
        t_ref[...] = jnp.dot(nf2, wsd_ref[...], preferred_element_type=_F32)
        ch = jnp.maximum(
            jnp.dot(nf2, wc1_ref[...], preferred_element_type=_F32)
            + bc1_ref[...], 0.0)
        p_ref[...] = (jnp.dot(ch, wc2_ref[...], preferred_element_type=_F32)
                      + bc2_ref[...])

    return pl.pallas_call(
        body,
        grid=(_N // _BN,),
        in_specs=[_rows(_BN, _ND)] + agg_specs
        + [_full((_ND, _ND)), _full((_ED, _ND)), _full((1, _ND)),
           _full((_ND, _ND)),
           _full((_ND, dh)), _full((1, dh)),
           _full((dh, 1)), _full((1, 1))],
        out_specs=[_rows(_BN, _ND), _rows(_BN, 1)],
        out_shape=[jax.ShapeDtypeStruct((_N, _ND), _F32),
                   jax.ShapeDtypeStruct((_N, 1), _F32)],
    )(nf, *aggs, Wn1, Wn2, bn.reshape(1, _ND), Wsd,
      Wc1, bc1.reshape(1, dh), Wc2, bc2.reshape(1, 1))


def _head(xin, W1, b1, W2, b2, block):
    rows, din = xin.shape
    dh = W1.shape[1]

    def body(x_ref, w1_ref, b1_ref, w2_ref, b2_ref, o_ref):
        h = jnp.maximum(
            jnp.dot(x_ref[...], w1_ref[...], preferred_element_type=_F32)
            + b1_ref[...], 0.0)
        o_ref[...] = (jnp.dot(h, w2_ref[...], preferred_element_type=_F32)
                      + b2_ref[...])

    return pl.pallas_call(
        body,
        grid=(rows // block,),
        in_specs=[_rows(block, din), _full((din, dh)), _full((1, dh)),
                  _full((dh, 1)), _full((1, 1))],
        out_specs=_rows(block, 1),
        out_shape=jax.ShapeDtypeStruct((rows, 1), _F32),
    )(xin, W1, b1.reshape(1, dh), W2, b2.reshape(1, 1))


def kernel(x, edge_attr, edge_index, node_types, params):
    p = params
    Wne, bne = p['node_emb'][0]
    Wee, bee = p['edge_emb'][0]
    (W1, b1), (W2, b2) = p['edge_mlp']
    Wn, bn = p['node_upd'][0]
    (Wc1, bc1), (Wc2, bc2) = p['node_cls']
    (We1, be1), (We2, be2) = p['edge_cls']

    Wsd = jnp.concatenate([W1[:_ND], W1[_ND:2 * _ND]], axis=1)
    W1e = W1[2 * _ND:]
    Wn1, Wn2 = Wn[:_ND], Wn[_ND:]

    _EA = 192000
    _EB = _E - _EA
    srcA = edge_index[0, :_EA].reshape(_NW, -1, _CH)
    dstA = edge_index[1, :_EA].reshape(_NW, -1, _CH)
    srcB = edge_index[0, _EA:].reshape(_NW, -1, _CH)
    dstB = edge_index[1, _EA:].reshape(_NW, -1, _CH)
    zeros = jnp.zeros((_NP, _ND), _F32)

    nf, tbl = _node_embed(x, Wne, bne, Wsd)
    efA = _edge_embed(edge_attr, Wee, bee, 0, _EA)
    efB = _edge_embed(edge_attr, Wee, bee, _EA, _EB)

    for step in range(2):
        gsA, gdA = _sc_gather(tbl, srcA, dstA)
        efA = _edge_mlp(gsA, gdA, efA, W1e, b1, W2, b2, out_pad=True)
        gsB, gdB = _sc_gather(tbl, srcB, dstB)
        efB = _edge_mlp(gsB, gdB, efB, W1e, b1, W2, b2, out_pad=True)
        aggA = _sc_segment_sum(efA, dstA, zeros)
        aggB = _sc_segment_sum(efB, dstB, zeros)
        aggs = [aggA[:_N], aggA[_NP:_NP + _N],
                aggB[:_N], aggB[_NP:_NP + _N]]
        if step == 0:
            nf, tbl = _node_update(nf, aggs, Wn1, Wn2, bn, Wsd)
        else:
            tbl, pred_node = _node_update(nf, aggs, Wn1, Wn2, bn, Wsd,
                                          head=(Wc1, bc1, Wc2, bc2))

    gsA, gdA = _sc_gather(tbl, srcA, dstA)
    peA = _edge_mlp_head(gsA, gdA, efA, W1e, b1, W2, b2, We1, be1, We2, be2)
    gsB, gdB = _sc_gather(tbl, srcB, dstB)
    peB = _edge_mlp_head(gsB, gdB, efB, W1e, b1, W2, b2, We1, be1, We2, be2)
    pred_edge = jnp.concatenate([peA, peB], axis=0)

    return (pred_edge[:, 0], pred_node[:, 0])


# speedup vs baseline: 1.0181x; 1.0181x over previous
import functools

import jax
import jax.numpy as jnp
from jax import lax
from jax.experimental import pallas as pl
from jax.experimental.pallas import tpu as pltpu
from jax.experimental.pallas import tpu_sc as plsc

_N = 10000
_E = 320000
_NODE_IN = 128
_EDGE_IN = 16
_ND = 128
_ED = 64

_NW = 32
_EPW = _E // _NW
_CH = 80
_NCH = _EPW // _CH
_NP = 10240
_RPT = _NP // 16

_F32 = jnp.float32


def _mesh():
    return plsc.VectorSubcoreMesh(core_axis_name="c", subcore_axis_name="s")


def _sc_gather(tbl, src3, dst3):
    nw, nch, ch = src3.shape
    epw = nch * ch
    ne = nw * epw

    @functools.partial(
        pl.kernel,
        mesh=_mesh(),
        out_type=[jax.ShapeDtypeStruct((ne, _ND), _F32),
                  jax.ShapeDtypeStruct((ne, _ND), _F32)],
        scratch_types=(
            [pltpu.VMEM((nch, ch), jnp.int32)] * 2
            + [pltpu.VMEM((ch, _ND), _F32)] * 6
            + [pltpu.SemaphoreType.DMA] * 12),
    )
    def k(tbl_hbm, src_hbm, dst_hbm, gs_hbm, gd_hbm,
          sidx, didx, bs0, bs1, bs2, bd0, bd1, bd2,
          g0, g1, g2, h0, h1, h2, w0, w1, w2, x0, x1, x2):
        wid = lax.axis_index("s") * 2 + lax.axis_index("c")
        pltpu.sync_copy(src_hbm.at[wid], sidx)
        pltpu.sync_copy(dst_hbm.at[wid], didx)
        bss, bds = (bs0, bs1, bs2), (bd0, bd1, bd2)
        gss, gds = (g0, g1, g2), (h0, h1, h2)
        wss, wds = (w0, w1, w2), (x0, x1, x2)

        def issue(c, j):
            pltpu.async_copy(tbl_hbm.at[sidx.at[c]], bss[j], gss[j])
            pltpu.async_copy(tbl_hbm.at[didx.at[c]], bds[j], gds[j])

        def wait_g(c, j):
            pltpu.make_async_copy(tbl_hbm.at[sidx.at[c]], bss[j],
                                  gss[j]).wait()
            pltpu.make_async_copy(tbl_hbm.at[didx.at[c]], bds[j],
                                  gds[j]).wait()

        def wr(c, j):
            base = wid * epw + c * ch
            pltpu.async_copy(bss[j], gs_hbm.at[pl.ds(base, ch)], wss[j])
            pltpu.async_copy(bds[j], gd_hbm.at[pl.ds(base, ch)], wds[j])

        def wait_w(c, j):
            base = wid * epw + c * ch
            pltpu.make_async_copy(bss[j], gs_hbm.at[pl.ds(base, ch)],
                                  wss[j]).wait()
            pltpu.make_async_copy(bds[j], gd_hbm.at[pl.ds(base, ch)],
                                  wds[j]).wait()

        issue(0, 0)
        issue(1, 1)
        wait_g(0, 0)
        wr(0, 0)
        issue(2, 2)
        wait_g(1, 1)
        wr(1, 1)
        wait_w(0, 0)
        issue(3, 0)

        def body(g, carry):
            for i in range(3):
                c = 3 * g + 2 + i
                j = (2 + i) % 3
                j2 = (1 + i) % 3

                wait_g(c, j)
                wr(c, j)

                @pl.when(c + 2 < nch)
                def _():
                    wait_w(c - 1, j2)
                    issue(c + 2, j2)

            return carry

        nloop = (nch - 2) // 3
        lax.fori_loop(0, nloop, body, 0)
        for c in range(3 * nloop + 2, nch):
            wait_g(c, c % 3)
            wr(c, c % 3)
        for c in range(nch - 3, nch):
            wait_w(c, c % 3)

    return k(tbl, src3, dst3)


def _sc_segment_sum(vals, dst3, zeros):
    nw, nch, ch = dst3.shape
    epw = nch * ch

    @functools.partial(
        pl.kernel,
        mesh=_mesh(),
        out_type=jax.ShapeDtypeStruct((2 * _NP, _ND), _F32),
        scratch_types=[pltpu.VMEM((nch, ch), jnp.int32),
                       pltpu.VMEM((ch, _ND), _F32),
                       pltpu.VMEM((ch, _ND), _F32),
                       pltpu.VMEM_SHARED((_NP, _ND), _F32),
                       pltpu.SemaphoreType.DMA,
                       pltpu.SemaphoreType.DMA],
    )
    def k(vals_hbm, dst_hbm, z_hbm, out_hbm, didx, bv0, bv1, acc, v0, v1):
        cid = lax.axis_index("c")
        sid = lax.axis_index("s")
        wid = sid * 2 + cid
        r0 = sid * _RPT
        pltpu.sync_copy(z_hbm.at[pl.ds(r0, _RPT)], acc.at[pl.ds(r0, _RPT)])
        pltpu.sync_copy(dst_hbm.at[wid], didx)
        plsc.subcore_barrier()
        bvs, vs = (bv0, bv1), (v0, v1)

        for j in range(2):
            pltpu.async_copy(
                vals_hbm.at[pl.ds(wid * epw + j * ch, ch)], bvs[j], vs[j])

        def step(c, j):
            base = wid * epw + c * ch
            pltpu.make_async_copy(
                vals_hbm.at[pl.ds(base, ch)], bvs[j], vs[j]).wait()
            pltpu.sync_copy(bvs[j], acc.at[didx.at[c]], add=True)

            @pl.when(c + 2 < nch)
            def _():
                pltpu.async_copy(
                    vals_hbm.at[pl.ds(base + 2 * ch, ch)], bvs[j], vs[j])

        def body(g, carry):
            step(2 * g, 0)
            step(2 * g + 1, 1)
            return carry

        lax.fori_loop(0, nch // 2, body, 0)
        if nch % 2:
            step(nch - 1, 0)
        plsc.subcore_barrier()
        pltpu.sync_copy(acc.at[pl.ds(r0, _RPT)],
                        out_hbm.at[pl.ds(cid * _NP + r0, _RPT)])

    return k(vals, dst3, zeros)


_BN = 2000
_BE = 4000


def _full(shape):
    return pl.BlockSpec(shape, lambda i: tuple(0 for _ in shape))


def _rows(block, width):
    return pl.BlockSpec((block, width), lambda i: (i, 0))


def _node_embed(x, Wne, bne, Wsd):
    def body(x_ref, w_ref, b_ref, wsd_ref, nf_ref, t_ref):
        nf = jnp.dot(x_ref[...], w_ref[...],
                     preferred_element_type=_F32) + b_ref[...]
        nf_ref[...] = nf
        t_ref[...] = jnp.dot(nf, wsd_ref[...], preferred_element_type=_F32)

    return pl.pallas_call(
        body,
        grid=(_N // _BN,),
        in_specs=[_rows(_BN, _NODE_IN), _full((_NODE_IN, _ND)),
                  _full((1, _ND)), _full((_ND, _ND))],
        out_specs=[_rows(_BN, _ND), _rows(_BN, _ND)],
        out_shape=[jax.ShapeDtypeStruct((_N, _ND), _F32),
                   jax.ShapeDtypeStruct((_N, _ND), _F32)],
    )(x, Wne, bne.reshape(1, _ND), Wsd)


def _edge_embed(ea, Wee, bee, off, ne):
    ob = off // _BE

    def body(a_ref, w_ref, b_ref, o_ref):
        o_ref[...] = jnp.dot(a_ref[...], w_ref[...],
                             preferred_element_type=_F32) + b_ref[...]

    return pl.pallas_call(
        body,
        grid=(ne // _BE,),
        in_specs=[pl.BlockSpec((_BE, _EDGE_IN), lambda i: (i + ob, 0)),
                  _full((_EDGE_IN, _ED)), _full((1, _ED))],
        out_specs=_rows(_BE, _ED),
        out_shape=jax.ShapeDtypeStruct((ne, _ED), _F32),
    )(ea, Wee, bee.reshape(1, _ED))


def _edge_mlp(gs, gd, ef, W1e, b1, W2, b2, out_pad):
    ne, ef_w = ef.shape
    out_w = _ND if out_pad else _ED

    def body(gs_ref, gd_ref, ef_ref, w1_ref, b1_ref, w2_ref, b2_ref, o_ref):
        if ef_w == _EDGE_IN:
            efv = ef_ref[...]
            prec = jax.lax.Precision.HIGHEST
        else:
            efv = ef_ref[...] if ef_w == _ED else ef_ref[:, :_ED]
            prec = jax.lax.Precision.DEFAULT
        h = (gs_ref[:, :_ED] + gd_ref[:, _ED:]
             + jnp.dot(efv, w1_ref[...], preferred_element_type=_F32,
                       precision=prec)
             + b1_ref[...])
        h = jnp.maximum(h, 0.0)
        o = jnp.dot(h, w2_ref[...], preferred_element_type=_F32) + b2_ref[...]
        o = jnp.maximum(o, 0.0)
        if out_pad:
            o_ref[...] = jnp.concatenate(
                [o, jnp.zeros((o.shape[0], _ND - _ED), _F32)], axis=1)
        else:
            o_ref[...] = o

    k1 = W1e.shape[0]
    return pl.pallas_call(
        body,
        grid=(ne // _BE,),
        in_specs=[_rows(_BE, _ND), _rows(_BE, _ND), _rows(_BE, ef_w),
                  _full((k1, _ED)), _full((1, _ED)),
                  _full((_ED, _ED)), _full((1, _ED))],
        out_specs=_rows(_BE, out_w),
        out_shape=jax.ShapeDtypeStruct((ne, out_w), _F32),
    )(gs, gd, ef, W1e, b1.reshape(1, _ED), W2, b2.reshape(1, _ED))


def _edge_mlp_head(gs, gd, ef, W1e, b1, W2, b2, We1, be1, We2, be2):
    ne, ef_w = ef.shape
    dh = We1.shape[1]

    def body(gs_ref, gd_ref, ef_ref, w1_ref, b1_ref, w2_ref, b2_ref,
             we1_ref, be1_ref, we2_ref, be2_ref, o_ref):
        efv = ef_ref[...] if ef_w == _ED else ef_ref[:, :_ED]
        h = (gs_ref[:, :_ED] + gd_ref[:, _ED:]
             + jnp.dot(efv, w1_ref[...], preferred_element_type=_F32)
             + b1_ref[...])
        h = jnp.maximum(h, 0.0)
        o = jnp.dot(h, w2_ref[...], preferred_element_type=_F32) + b2_ref[...]
        o = jnp.maximum(o, 0.0)
        ch = jnp.maximum(
            jnp.dot(o, we1_ref[...], preferred_element_type=_F32)
            + be1_ref[...], 0.0)
        o_ref[...] = (jnp.dot(ch, we2_ref[...], preferred_element_type=_F32)
                      + be2_ref[...])

    return pl.pallas_call(
        body,
        grid=(ne // _BE,),
        in_specs=[_rows(_BE, _ND), _rows(_BE, _ND), _rows(_BE, ef_w),
                  _full((_ED, _ED)), _full((1, _ED)),
                  _full((_ED, _ED)), _full((1, _ED)),
                  _full((_ED, dh)), _full((1, dh)),
                  _full((dh, 1)), _full((1, 1))],
        out_specs=_rows(_BE, 1),
        out_shape=jax.ShapeDtypeStruct((ne, 1), _F32),
    )(gs, gd, ef, W1e, b1.reshape(1, _ED), W2, b2.reshape(1, _ED),
      We1, be1.reshape(1, dh), We2, be2.reshape(1, 1))


def _node_update(nf, aggs, Wn1, Wn2, bn, Wsd, head=None):
    na = len(aggs)
    agg_specs = [_rows(_BN, _ND)] * na

    def agg_sum(arefs):
        s = arefs[0][:, :_ED]
        for a in arefs[1:]:
            s = s + a[:, :_ED]
        return s

    if head is None:
        def body(*refs):
            nf_ref = refs[0]
            arefs = refs[1:1 + na]
            wn1_ref, wn2_ref, b_ref, wsd_ref, nf2_ref, t_ref = refs[1 + na:]
            h = (jnp.dot(nf_ref[...], wn1_ref[...],
                         preferred_element_type=_F32)
                 + jnp.dot(agg_sum(arefs), wn2_ref[...],
                           preferred_element_type=_F32)
                 + b_ref[...])
            nf2 = jnp.maximum(h, 0.0)
            nf2_ref[...] = nf2
            t_ref[...] = jnp.dot(nf2, wsd_ref[...],
                                 preferred_element_type=_F32)

        return pl.pallas_call(
            body,
            grid=(_N // _BN,),
            in_specs=[_rows(_BN, _ND)] + agg_specs
            + [_full((_ND, _ND)), _full((_ED, _ND)), _full((1, _ND)),
               _full((_ND, _ND))],
            out_specs=[_rows(_BN, _ND), _rows(_BN, _ND)],
            out_shape=[jax.ShapeDtypeStruct((_N, _ND), _F32),
                       jax.ShapeDtypeStruct((_N, _ND), _F32)],
        )(nf, *aggs, Wn1, Wn2, bn.reshape(1, _ND), Wsd)

    Wc1, bc1, Wc2, bc2 = head
    dh = Wc1.shape[1]

    def body(*refs):
        nf_ref = refs[0]
        arefs = refs[1:1 + na]
        (wn1_ref, wn2_ref, b_ref, wsd_ref, wc1_ref, bc1_ref, wc2_ref,
         bc2_ref, t_ref, p_ref) = refs[1 + na:]
        h = (jnp.dot(nf_ref[...], wn1_ref[...], preferred_element_type=_F32)
             + jnp.dot(agg_sum(arefs), wn2_ref[...],
                       preferred_element_type=_F32)
             + b_ref[...])
        nf2 = jnp.maximum(h, 0.0)
        t_ref[...] = jnp.dot(nf2, wsd_ref[...], preferred_element_type=_F32)
        ch = jnp.maximum(
            jnp.dot(nf2, wc1_ref[...], preferred_element_type=_F32)
            + bc1_ref[...], 0.0)
        p_ref[...] = (jnp.dot(ch, wc2_ref[...], preferred_element_type=_F32)
                      + bc2_ref[...])

    return pl.pallas_call(
        body,
        grid=(_N // _BN,),
        in_specs=[_rows(_BN, _ND)] + agg_specs
        + [_full((_ND, _ND)), _full((_ED, _ND)), _full((1, _ND)),
           _full((_ND, _ND)),
           _full((_ND, dh)), _full((1, dh)),
           _full((dh, 1)), _full((1, 1))],
        out_specs=[_rows(_BN, _ND), _rows(_BN, 1)],
        out_shape=[jax.ShapeDtypeStruct((_N, _ND), _F32),
                   jax.ShapeDtypeStruct((_N, 1), _F32)],
    )(nf, *aggs, Wn1, Wn2, bn.reshape(1, _ND), Wsd,
      Wc1, bc1.reshape(1, dh), Wc2, bc2.reshape(1, 1))


def _head(xin, W1, b1, W2, b2, block):
    rows, din = xin.shape
    dh = W1.shape[1]

    def body(x_ref, w1_ref, b1_ref, w2_ref, b2_ref, o_ref):
        h = jnp.maximum(
            jnp.dot(x_ref[...], w1_ref[...], preferred_element_type=_F32)
            + b1_ref[...], 0.0)
        o_ref[...] = (jnp.dot(h, w2_ref[...], preferred_element_type=_F32)
                      + b2_ref[...])

    return pl.pallas_call(
        body,
        grid=(rows // block,),
        in_specs=[_rows(block, din), _full((din, dh)), _full((1, dh)),
                  _full((dh, 1)), _full((1, 1))],
        out_specs=_rows(block, 1),
        out_shape=jax.ShapeDtypeStruct((rows, 1), _F32),
    )(xin, W1, b1.reshape(1, dh), W2, b2.reshape(1, 1))


def kernel(x, edge_attr, edge_index, node_types, params):
    p = params
    Wne, bne = p['node_emb'][0]
    Wee, bee = p['edge_emb'][0]
    (W1, b1), (W2, b2) = p['edge_mlp']
    Wn, bn = p['node_upd'][0]
    (Wc1, bc1), (Wc2, bc2) = p['node_cls']
    (We1, be1), (We2, be2) = p['edge_cls']

    Wsd = jnp.concatenate([W1[:_ND], W1[_ND:2 * _ND]], axis=1)
    W1e = W1[2 * _ND:]
    Wn1, Wn2 = Wn[:_ND], Wn[_ND:]

    src3 = edge_index[0].reshape(_NW, _NCH, _CH)
    dst3 = edge_index[1].reshape(_NW, _NCH, _CH)
    zeros = jnp.zeros((_NP, _ND), _F32)

    nf, tbl = _node_embed(x, Wne, bne, Wsd)
    ef = _edge_embed(edge_attr, Wee, bee, 0, _E)

    gs, gd = _sc_gather(tbl, src3, dst3)
    ef = _edge_mlp(gs, gd, ef, W1e, b1, W2, b2, out_pad=True)
    aggp = _sc_segment_sum(ef, dst3, zeros)
    nf, tbl = _node_update(nf, [aggp[:_N], aggp[_NP:_NP + _N]],
                           Wn1, Wn2, bn, Wsd)
    gs, gd = _sc_gather(tbl, src3, dst3)
    ef = _edge_mlp(gs, gd, ef, W1e, b1, W2, b2, out_pad=True)
    aggp = _sc_segment_sum(ef, dst3, zeros)
    tbl, pred_node = _node_update(nf, [aggp[:_N], aggp[_NP:_NP + _N]],
                                  Wn1, Wn2, bn, Wsd,
                                  head=(Wc1, bc1, Wc2, bc2))
    gs, gd = _sc_gather(tbl, src3, dst3)
    pred_edge = _edge_mlp_head(gs, gd, ef, W1e, b1, W2, b2,
                               We1, be1, We2, be2)

    return (pred_edge[:, 0], pred_node[:, 0])
